# baseline (speedup 1.0000x reference)
import functools

import jax
import jax.numpy as jnp
from jax import lax
from jax.experimental import pallas as pl
from jax.experimental.pallas import tpu as pltpu
from jax.experimental.pallas import tpu_sc as plsc

N_FRAG = 1600000
N_SEG = 100000
N_FREQ = 10

NW = 32
CHUNK = 128
CPW = 392
PAD_N = NW * CPW * CHUNK
ROWS = PAD_N // CHUNK
FIRE = 14
OUTER = 28

S1_ROWS = 1250
S1_OROWS = 1256
S1_COLS = 2560
S1_BLK = 256
S1_OBLK = 128
RS_GRID = 157


def _embed_body(p_ref, x_ref, o_ref):
    x = x_ref[...]
    m = lax.broadcasted_iota(jnp.int32, (S1_BLK, S1_OBLK), 0)
    l = lax.broadcasted_iota(jnp.int32, (S1_BLK, S1_OBLK), 1)
    pe = jnp.where(m == 2 * l, 1.0, 0.0).astype(jnp.float32)
    po = jnp.where(m == 2 * l + 1, 1.0, 0.0).astype(jnp.float32)
    xe = lax.dot(x, pe, precision=lax.Precision.HIGHEST)
    xo = lax.dot(x, po, precision=lax.Precision.HIGHEST)
    acc = jnp.zeros_like(xe)
    for j in range(N_FREQ):
        f = p_ref[0, j]
        acc = acc + p_ref[1, j] * jnp.sin(xe * f + p_ref[2, j])
        acc = acc + p_ref[3, j] * jnp.sin(xo * f + p_ref[4, j])
    o_ref[0:S1_ROWS, :] = acc
    o_ref[pl.ds(S1_ROWS, S1_OROWS - S1_ROWS), :] = jnp.zeros(
        (S1_OROWS - S1_ROWS, S1_OBLK), jnp.float32)


_embed = pl.pallas_call(
    _embed_body,
    grid=(S1_COLS // S1_BLK,),
    in_specs=[
        pl.BlockSpec(memory_space=pltpu.SMEM),
        pl.BlockSpec((S1_ROWS, S1_BLK), lambda i: (0, i)),
    ],
    out_specs=pl.BlockSpec((S1_OROWS, S1_OBLK), lambda i: (0, i)),
    out_shape=jax.ShapeDtypeStruct((S1_OROWS, S1_COLS // 2), jnp.float32),
)


def _reshape_body(x_ref, o_ref):
    o_ref[...] = x_ref[...].reshape(80, 128)


_reshape = pl.pallas_call(
    _reshape_body,
    grid=(RS_GRID,),
    in_specs=[pl.BlockSpec((8, S1_COLS // 2), lambda i: (i, 0))],
    out_specs=pl.BlockSpec((80, CHUNK), lambda i: (i, 0)),
    out_shape=jax.ShapeDtypeStruct((RS_GRID * 80, CHUNK), jnp.float32),
)


def _scatter_body(seg_hbm, val_hbm, zero_hbm, out_hbm,
                  ids_v, vals_v, acc, sem_in, sem_sc):
    c = lax.axis_index("c")
    s = lax.axis_index("s")
    wid = s * 2 + c
    row0 = wid * CPW

    cp_i = pltpu.async_copy(seg_hbm.at[pl.ds(row0, CPW)], ids_v, sem_in)
    cp_v = pltpu.async_copy(val_hbm.at[pl.ds(row0, CPW)], vals_v, sem_in)

    @pl.when(s == 0)
    def _():
        pltpu.sync_copy(zero_hbm, acc)

    cp_i.wait()
    cp_v.wait()
    plsc.subcore_barrier()

    def outer_body(o, carry):
        descs = []
        for k in range(FIRE):
            j = o * FIRE + k
            descs.append(
                pltpu.async_copy(vals_v.at[j], acc.at[ids_v.at[j]],
                                 sem_sc, add=True))
        for d in descs:
            d.wait()
        return carry

    lax.fori_loop(0, OUTER, outer_body, 0)
    plsc.subcore_barrier()

    @pl.when(s == 0)
    def _():
        pltpu.sync_copy(acc, out_hbm.at[c])


@functools.cache
def _make_scatter():
    return functools.partial(
        pl.kernel,
        out_type=jax.ShapeDtypeStruct((2, N_SEG), jnp.float32),
        mesh=plsc.VectorSubcoreMesh(core_axis_name="c", subcore_axis_name="s"),
        scratch_types=[
            pltpu.VMEM((CPW, CHUNK), jnp.int32),
            pltpu.VMEM((CPW, CHUNK), jnp.float32),
            pltpu.VMEM_SHARED((N_SEG,), jnp.float32),
            pltpu.SemaphoreType.DMA,
            pltpu.SemaphoreType.DMA,
        ],
    )(_scatter_body)


def _combine_body(p_ref, b_ref, o_ref):
    o_ref[...] = p_ref[0] + p_ref[1] + b_ref[...]


def _combine(partials, bias_row, cell_n_static, gene_n_static):
    return pl.pallas_call(
        _combine_body,
        out_shape=jax.ShapeDtypeStruct((cell_n_static, gene_n_static),
                                       jnp.float32),
    )(partials, bias_row)


def kernel(fragment_coordinates, fragment_cellxgene_ix, cell_n, gene_n,
           gene_ix, weight1, bias1):
    gene_n_static = gene_ix.shape[0]
    cell_n_static = 1000
    num_segments = cell_n_static * gene_n_static

    i = jnp.arange(1, N_FREQ + 1, dtype=jnp.float32)
    freqs = 1.0 / (100.0 ** (2.0 * i / N_FREQ))
    w = weight1.reshape(2, N_FREQ, 2)
    a = w[:, :, 0]
    b = w[:, :, 1]
    r = jnp.sqrt(a * a + b * b)
    phi = jnp.arctan2(b, a)
    params = jnp.stack([freqs, r[0], phi[0], r[1], phi[1]]).astype(jnp.float32)

    one = bias1[0] * 0.0 + 1.0
    x = fragment_coordinates.reshape(S1_ROWS, S1_COLS) * one
    vals = _reshape(_embed(params, x))

    seg = (fragment_cellxgene_ix
           + (cell_n * gene_n - num_segments)).astype(jnp.int32)
    n_pad = PAD_N - N_FRAG
    pad_ids = jnp.arange(n_pad, dtype=jnp.int32) % num_segments
    seg_p = jnp.concatenate([seg, pad_ids]).reshape(ROWS, CHUNK)

    partials = _make_scatter()(seg_p, vals, jnp.zeros((N_SEG,), jnp.float32))

    bias_row = bias1[gene_ix].reshape(1, gene_n_static).astype(jnp.float32)
    return _combine(partials.reshape(2, cell_n_static, gene_n_static),
                    bias_row, cell_n_static, gene_n_static)


# speedup vs baseline: 12.0418x; 12.0418x over previous
import functools

import jax
import jax.numpy as jnp
from jax import lax
from jax.experimental import pallas as pl
from jax.experimental.pallas import tpu as pltpu
from jax.experimental.pallas import tpu_sc as plsc

N_FRAG = 1600000
N_SEG = 100000
N_FREQ = 10

NW = 32
CHUNK = 128
CPW = 392
PAD_N = NW * CPW * CHUNK
ROWS = PAD_N // CHUNK
FIRE = 14
OUTER = 28

S1_ROWS = 1250
S1_COLS = 1280
S1_BLK = 128

_INV_PI = 0.3183098861837907
_PI = 3.141592653589793
_S2 = -0.16666666641626524
_S4 = 0.008333329385889463
_S6 = -0.00019840874
_S8 = 2.7525562e-06


def _fast_sin(t):
    k = jnp.round(t * _INV_PI)
    r = t - k * _PI
    odd = (k.astype(jnp.int32) & 1) << 31
    r2 = r * r
    p = _S8
    p = p * r2 + _S6
    p = p * r2 + _S4
    p = p * r2 + _S2
    s = r + r * (r2 * p)
    return lax.bitcast_convert_type(
        lax.bitcast_convert_type(s, jnp.int32) ^ odd, jnp.float32)


def _embed_body(p_ref, x0_ref, x1_ref, o_ref):
    x0 = x0_ref[...]
    x1 = x1_ref[...]
    acc = jnp.zeros_like(x0)
    for j in range(N_FREQ):
        f = p_ref[0, j]
        acc = acc + p_ref[1, j] * _fast_sin(x0 * f + p_ref[2, j])
        acc = acc + p_ref[3, j] * _fast_sin(x1 * f + p_ref[4, j])
    o_ref[...] = acc


_embed = pl.pallas_call(
    _embed_body,
    grid=(S1_COLS // S1_BLK,),
    in_specs=[
        pl.BlockSpec(memory_space=pltpu.SMEM),
        pl.BlockSpec((S1_ROWS, S1_BLK), lambda i: (0, i)),
        pl.BlockSpec((S1_ROWS, S1_BLK), lambda i: (0, i)),
    ],
    out_specs=pl.BlockSpec((S1_ROWS, S1_BLK), lambda i: (0, i)),
    out_shape=jax.ShapeDtypeStruct((S1_ROWS, S1_COLS), jnp.float32),
)


def _scatter_body(seg_hbm, val_hbm, zero_hbm, out_hbm,
                  ids_v, vals_v, acc, sem_in, sem_sc):
    c = lax.axis_index("c")
    s = lax.axis_index("s")
    wid = s * 2 + c
    row0 = wid * CPW

    cp_i = pltpu.async_copy(seg_hbm.at[pl.ds(row0, CPW)], ids_v, sem_in)
    cp_v = pltpu.async_copy(val_hbm.at[pl.ds(row0, CPW)], vals_v, sem_in)

    @pl.when(s == 0)
    def _():
        pltpu.sync_copy(zero_hbm, acc)

    cp_i.wait()
    cp_v.wait()
    plsc.subcore_barrier()

    def outer_body(o, carry):
        descs = []
        for k in range(FIRE):
            j = o * FIRE + k
            descs.append(
                pltpu.async_copy(vals_v.at[j], acc.at[ids_v.at[j]],
                                 sem_sc, add=True))
        for d in descs:
            d.wait()
        return carry

    lax.fori_loop(0, OUTER, outer_body, 0)
    plsc.subcore_barrier()

    @pl.when(s == 0)
    def _():
        pltpu.sync_copy(acc, out_hbm.at[c])


@functools.cache
def _make_scatter():
    return functools.partial(
        pl.kernel,
        out_type=jax.ShapeDtypeStruct((2, N_SEG), jnp.float32),
        mesh=plsc.VectorSubcoreMesh(core_axis_name="c", subcore_axis_name="s"),
        scratch_types=[
            pltpu.VMEM((CPW, CHUNK), jnp.int32),
            pltpu.VMEM((CPW, CHUNK), jnp.float32),
            pltpu.VMEM_SHARED((N_SEG,), jnp.float32),
            pltpu.SemaphoreType.DMA,
            pltpu.SemaphoreType.DMA,
        ],
    )(_scatter_body)


def _combine_body(p_ref, b_ref, o_ref):
    o_ref[...] = p_ref[0] + p_ref[1] + b_ref[...]


def _combine(partials, bias_row, cell_n_static, gene_n_static):
    return pl.pallas_call(
        _combine_body,
        out_shape=jax.ShapeDtypeStruct((cell_n_static, gene_n_static),
                                       jnp.float32),
    )(partials, bias_row)


def kernel(fragment_coordinates, fragment_cellxgene_ix, cell_n, gene_n,
           gene_ix, weight1, bias1):
    gene_n_static = gene_ix.shape[0]
    cell_n_static = 1000
    num_segments = cell_n_static * gene_n_static

    i = jnp.arange(1, N_FREQ + 1, dtype=jnp.float32)
    freqs = 1.0 / (100.0 ** (2.0 * i / N_FREQ))
    w = weight1.reshape(2, N_FREQ, 2)
    a = w[:, :, 0]
    b = w[:, :, 1]
    r = jnp.sqrt(a * a + b * b)
    phi = jnp.arctan2(b, a)
    params = jnp.stack([freqs, r[0], phi[0], r[1], phi[1]]).astype(jnp.float32)

    xt = fragment_coordinates.T.reshape(2, S1_ROWS, S1_COLS)
    vals = _embed(params, xt[0], xt[1]).reshape(-1)

    seg = (fragment_cellxgene_ix
           + (cell_n * gene_n - num_segments)).astype(jnp.int32)
    n_pad = PAD_N - N_FRAG
    pad_ids = jnp.arange(n_pad, dtype=jnp.int32) % num_segments
    seg_p = jnp.concatenate([seg, pad_ids]).reshape(ROWS, CHUNK)
    val_p = jnp.concatenate(
        [vals, jnp.zeros((n_pad,), jnp.float32)]).reshape(ROWS, CHUNK)

    partials = _make_scatter()(seg_p, val_p, jnp.zeros((N_SEG,), jnp.float32))

    bias_row = bias1[gene_ix].reshape(1, gene_n_static).astype(jnp.float32)
    return _combine(partials.reshape(2, cell_n_static, gene_n_static),
                    bias_row, cell_n_static, gene_n_static)
